# baseline (speedup 1.0000x reference)
import jax
import jax.numpy as jnp
from jax.experimental import pallas as pl

_B = 4096
_F = 100
_D = 128
_T = _F + 1
_BB = 128


def _body(x_ref, w_ref, b_ref, o_ref):
    o_ref[...] = x_ref[...] * w_ref[...][None] + b_ref[...][None]


def kernel(x, weight, bias, cls_token):
    xpad = jnp.concatenate([jnp.ones((_B, 1), jnp.float32), x], axis=1)
    x3 = xpad[:, :, None]
    wpad = jnp.concatenate([cls_token.reshape(1, _D), weight], axis=0)
    bpad = jnp.concatenate([jnp.zeros((1, _D), jnp.float32), bias], axis=0)
    return pl.pallas_call(
        _body,
        grid=(_B // _BB,),
        in_specs=[
            pl.BlockSpec((_BB, _T, 1), lambda i: (i, 0, 0)),
            pl.BlockSpec((_T, _D), lambda i: (0, 0)),
            pl.BlockSpec((_T, _D), lambda i: (0, 0)),
        ],
        out_specs=pl.BlockSpec((_BB, _T, _D), lambda i: (i, 0, 0)),
        out_shape=jax.ShapeDtypeStruct((_B, _T, _D), jnp.float32),
    )(x3, wpad, bpad)


# speedup vs baseline: 1.8510x; 1.8510x over previous
import jax
import jax.numpy as jnp
from jax import lax
from jax.experimental import pallas as pl
from jax.experimental.pallas import tpu as pltpu

_B = 4096
_F = 100
_D = 128
_T = _F + 1
_BB = 128
_NSTEP = _B // _BB
_NBUF = 4


def _body(x_ref, w_ref, b_ref, o_hbm, scratch, sems):
    i = pl.program_id(0)
    buf = lax.rem(i, _NBUF)

    @pl.when(i >= _NBUF)
    def _wait_prev():
        pltpu.make_async_copy(
            scratch.at[buf],
            o_hbm.at[pl.ds((i - _NBUF) * _BB, _BB)],
            sems.at[buf],
        ).wait()

    scratch[buf] = x_ref[...][:, :, None] * w_ref[...][None] + b_ref[...][None]
    pltpu.make_async_copy(
        scratch.at[buf], o_hbm.at[pl.ds(i * _BB, _BB)], sems.at[buf]
    ).start()

    @pl.when(i == _NSTEP - 1)
    def _drain():
        for j in range(_NBUF):
            step = _NSTEP - _NBUF + j
            pltpu.make_async_copy(
                scratch.at[step % _NBUF],
                o_hbm.at[pl.ds(step * _BB, _BB)],
                sems.at[step % _NBUF],
            ).wait()


def kernel(x, weight, bias, cls_token):
    xpad = jnp.concatenate([jnp.ones((_B, 1), jnp.float32), x], axis=1)
    wpad = jnp.concatenate([cls_token.reshape(1, _D), weight], axis=0)
    bpad = jnp.concatenate([jnp.zeros((1, _D), jnp.float32), bias], axis=0)
    return pl.pallas_call(
        _body,
        grid=(_NSTEP,),
        in_specs=[
            pl.BlockSpec((_BB, _T), lambda i: (i, 0)),
            pl.BlockSpec((_T, _D), lambda i: (0, 0)),
            pl.BlockSpec((_T, _D), lambda i: (0, 0)),
        ],
        out_specs=pl.BlockSpec(memory_space=pltpu.MemorySpace.HBM),
        out_shape=jax.ShapeDtypeStruct((_B, _T, _D), jnp.float32),
        scratch_shapes=[
            pltpu.VMEM((_NBUF, _BB, _T, _D), jnp.float32),
            pltpu.SemaphoreType.DMA((_NBUF,)),
        ],
        compiler_params=pltpu.CompilerParams(
            dimension_semantics=("arbitrary",),
        ),
    )(xpad, wpad, bpad)
